# baseline (speedup 1.0000x reference)
import functools

import jax
import jax.numpy as jnp
from jax import lax
from jax.experimental import pallas as pl
from jax.experimental.pallas import tpu as pltpu
from jax.experimental.pallas import tpu_sc as plsc


def _sc_impl(x, pos_table):
    B, S, D = x.shape
    NC, NS = 2, 16
    NW = NC * NS
    SPW = S // NW
    CH = 8
    NBUF = 3
    T = SPW // CH

    mesh = plsc.VectorSubcoreMesh(core_axis_name="c", subcore_axis_name="s")

    @functools.partial(
        pl.kernel,
        mesh=mesh,
        out_type=jax.ShapeDtypeStruct((B, S, D), jnp.float32),
        scratch_types=(
            [pltpu.VMEM((CH, D), jnp.float32)] * NBUF
            + [pltpu.VMEM((B, CH, D), jnp.float32)] * NBUF
            + [pltpu.SemaphoreType.DMA] * NBUF
            + [pltpu.SemaphoreType.DMA] * NBUF
            + [pltpu.SemaphoreType.DMA] * NBUF
        ),
    )
    def k(x_hbm, p_hbm, o_hbm, *refs):
        pb = refs[0:NBUF]
        xb = refs[NBUF:2 * NBUF]
        sp = refs[2 * NBUF:3 * NBUF]
        si = refs[3 * NBUF:4 * NBUF]
        so = refs[4 * NBUF:5 * NBUF]
        wid = lax.axis_index("s") * NC + lax.axis_index("c")
        s0 = wid * SPW

        def srow(t):
            return s0 + t * CH

        def pos_copy(t):
            return pltpu.make_async_copy(
                p_hbm.at[pl.ds(srow(t), CH)], pb[t % NBUF], sp[t % NBUF])

        def in_copies(t):
            return [
                pltpu.make_async_copy(
                    x_hbm.at[b, pl.ds(srow(t), CH)], xb[t % NBUF].at[b],
                    si[t % NBUF])
                for b in range(B)
            ]

        def out_copies(t):
            return [
                pltpu.make_async_copy(
                    xb[t % NBUF].at[b], o_hbm.at[b, pl.ds(srow(t), CH)],
                    so[t % NBUF])
                for b in range(B)
            ]

        def start_in(t):
            pos_copy(t).start()
            for c in in_copies(t):
                c.start()

        start_in(0)
        start_in(1)
        for t in range(T):
            if t + 2 < T:
                if t + 2 >= NBUF:
                    for c in out_copies(t + 2 - NBUF):
                        c.wait()
                start_in(t + 2)
            pos_copy(t).wait()
            for c in in_copies(t):
                c.wait()
            xbuf = xb[t % NBUF]
            pbuf = pb[t % NBUF]

            def row_body(r, _, xbuf=xbuf, pbuf=pbuf):
                @plsc.parallel_loop(0, D, step=16, unroll=4)
                def _vec_body(c):
                    v = pbuf[r, pl.ds(c, 16)]
                    for b in range(B):
                        plsc.addupdate(xbuf.at[b, r, pl.ds(c, 16)], v)
                return 0

            lax.fori_loop(0, CH, row_body, 0)
            for c in out_copies(t):
                c.start()
        for t in range(max(0, T - NBUF), T):
            for c in out_copies(t):
                c.wait()

    return k(x, pos_table)


def kernel(x, pos_table):
    return _sc_impl(x, pos_table)
